# initial kernel scaffold (unmeasured)
import jax
import jax.numpy as jnp
from jax import lax
from jax.experimental import pallas as pl
from jax.experimental.pallas import tpu as pltpu


def kernel(
    x,
):
    def body(*refs):
        pass

    out_shape = jax.ShapeDtypeStruct(..., jnp.float32)
    return pl.pallas_call(body, out_shape=out_shape)(...)


# baseline (device time: 20678 ns/iter reference)
import jax
import jax.numpy as jnp
from jax import lax
from jax.experimental import pallas as pl
from jax.experimental.pallas import tpu as pltpu

N_DEV = 8


def kernel(x):
    m_per, n = x.shape

    def body(x_ref, out_ref, gather_ref, send_sems, recv_sems):
        my = lax.axis_index("i")

        xv = x_ref[:, :]
        vmax = jnp.max(xv, axis=0)
        rows = lax.broadcasted_iota(jnp.int32, (m_per, n), 0)
        loc = jnp.min(jnp.where(xv == vmax[None, :], rows, m_per), axis=0)
        gidx = (my * m_per + loc).astype(jnp.float32)
        partial = jnp.stack([vmax, gidx], axis=0)
        gather_ref[pl.ds(my, 1)] = partial[None]

        sends = []
        for off in range(1, N_DEV):
            dst = lax.rem(my + off, N_DEV)
            rdma = pltpu.make_async_remote_copy(
                src_ref=gather_ref.at[my],
                dst_ref=gather_ref.at[my],
                send_sem=send_sems.at[off - 1],
                recv_sem=recv_sems.at[my],
                device_id=(dst,),
                device_id_type=pl.DeviceIdType.MESH,
            )
            rdma.start()
            sends.append(rdma)

        for off in range(1, N_DEV):
            src = lax.rem(my - off + N_DEV, N_DEV)
            recv = pltpu.make_async_remote_copy(
                src_ref=gather_ref.at[my],
                dst_ref=gather_ref.at[src],
                send_sem=send_sems.at[0],
                recv_sem=recv_sems.at[src],
                device_id=(src,),
                device_id_type=pl.DeviceIdType.MESH,
            )
            recv.wait_recv()

        for rdma in sends:
            rdma.wait_send()

        allp = gather_ref[:, :, :]
        vals = allp[:, 0, :]
        idxs = allp[:, 1, :]
        gmax = jnp.max(vals, axis=0)
        big = jnp.float32(3.0e38)
        gi = jnp.min(jnp.where(vals == gmax[None, :], idxs, big), axis=0)
        out_ref[0, :] = gmax
        out_ref[1, :] = gi

    return pl.pallas_call(
        body,
        out_shape=jax.ShapeDtypeStruct((2, n), jnp.float32),
        in_specs=[pl.BlockSpec(memory_space=pltpu.VMEM)],
        out_specs=pl.BlockSpec(memory_space=pltpu.VMEM),
        scratch_shapes=[
            pltpu.VMEM((N_DEV, 2, n), jnp.float32),
            pltpu.SemaphoreType.DMA((N_DEV - 1,)),
            pltpu.SemaphoreType.DMA((N_DEV,)),
        ],
    )(x)


# device time: 20523 ns/iter; 1.0076x vs baseline; 1.0076x over previous
import jax
import jax.numpy as jnp
from jax import lax
from jax.experimental import pallas as pl
from jax.experimental.pallas import tpu as pltpu

N_DEV = 8
N_BLK = 8


def kernel(x):
    m_per, n = x.shape
    blk = m_per // N_BLK

    def body(x_ref, out_ref, run_ref, gather_ref, send_sems, recv_sems):
        my = lax.axis_index("i")
        i = pl.program_id(0)

        xv = x_ref[:, :]
        cmax = jnp.max(xv, axis=0)
        rows = lax.broadcasted_iota(jnp.int32, (blk, n), 0)
        loc = jnp.min(jnp.where(xv == cmax[None, :], rows, blk), axis=0)
        cidx = (my * m_per + i * blk + loc).astype(jnp.float32)

        @pl.when(i == 0)
        def _():
            run_ref[0, :] = cmax
            run_ref[1, :] = cidx

        @pl.when(i > 0)
        def _():
            better = cmax > run_ref[0, :]
            run_ref[0, :] = jnp.where(better, cmax, run_ref[0, :])
            run_ref[1, :] = jnp.where(better, cidx, run_ref[1, :])

        @pl.when(i == N_BLK - 1)
        def _():
            gather_ref[pl.ds(my, 1)] = run_ref[:, :][None]

            sends = []
            for off in range(1, N_DEV):
                dst = lax.rem(my + off, N_DEV)
                rdma = pltpu.make_async_remote_copy(
                    src_ref=gather_ref.at[my],
                    dst_ref=gather_ref.at[my],
                    send_sem=send_sems.at[off - 1],
                    recv_sem=recv_sems.at[my],
                    device_id=(dst,),
                    device_id_type=pl.DeviceIdType.MESH,
                )
                rdma.start()
                sends.append(rdma)

            for off in range(1, N_DEV):
                src = lax.rem(my - off + N_DEV, N_DEV)
                recv = pltpu.make_async_remote_copy(
                    src_ref=gather_ref.at[my],
                    dst_ref=gather_ref.at[src],
                    send_sem=send_sems.at[0],
                    recv_sem=recv_sems.at[src],
                    device_id=(src,),
                    device_id_type=pl.DeviceIdType.MESH,
                )
                recv.wait_recv()

            for rdma in sends:
                rdma.wait_send()

            vals = gather_ref[:, 0, :]
            idxs = gather_ref[:, 1, :]
            gmax = jnp.max(vals, axis=0)
            big = jnp.float32(3.0e38)
            gi = jnp.min(jnp.where(vals == gmax[None, :], idxs, big), axis=0)
            out_ref[0, :] = gmax
            out_ref[1, :] = gi

    return pl.pallas_call(
        body,
        grid=(N_BLK,),
        out_shape=jax.ShapeDtypeStruct((2, n), jnp.float32),
        in_specs=[
            pl.BlockSpec((blk, n), lambda i: (i, 0), memory_space=pltpu.VMEM)
        ],
        out_specs=pl.BlockSpec((2, n), lambda i: (0, 0), memory_space=pltpu.VMEM),
        scratch_shapes=[
            pltpu.VMEM((2, n), jnp.float32),
            pltpu.VMEM((N_DEV, 2, n), jnp.float32),
            pltpu.SemaphoreType.DMA((N_DEV - 1,)),
            pltpu.SemaphoreType.DMA((N_DEV,)),
        ],
    )(x)


# device time: 16480 ns/iter; 1.2547x vs baseline; 1.2453x over previous
import jax
import jax.numpy as jnp
from jax import lax
from jax.experimental import pallas as pl
from jax.experimental.pallas import tpu as pltpu

N_DEV = 8
N_BLK = 8
SUB = 8


def kernel(x):
    m_per, n = x.shape
    blk = m_per // N_BLK
    g_per_blk = blk // SUB

    def body(x_ref, out_ref, rv_ref, ri_ref, gather_ref, send_sems, recv_sems):
        my = lax.axis_index("i")
        i = pl.program_id(0)

        @pl.when(i == 0)
        def _():
            barrier_sem = pltpu.get_barrier_semaphore()
            for off in range(1, N_DEV):
                dst = lax.rem(my + off, N_DEV)
                pl.semaphore_signal(
                    barrier_sem, inc=1, device_id=(dst,),
                    device_id_type=pl.DeviceIdType.MESH,
                )
            rv_ref[:, :] = jnp.full((SUB, n), -jnp.inf, jnp.float32)
            ri_ref[:, :] = jnp.zeros((SUB, n), jnp.int32)

        rv = rv_ref[:, :]
        ri = ri_ref[:, :]
        for g in range(g_per_blk):
            seg = x_ref[g * SUB:(g + 1) * SUB, :]
            gid = i * g_per_blk + g
            better = seg > rv
            rv = jnp.where(better, seg, rv)
            ri = jnp.where(better, gid, ri)
        rv_ref[:, :] = rv
        ri_ref[:, :] = ri

        @pl.when(i == N_BLK - 1)
        def _():
            sub = lax.broadcasted_iota(jnp.int32, (SUB, n), 0)
            grow = (my * m_per + ri * SUB + sub).astype(jnp.float32)
            vmax = jnp.max(rv, axis=0)
            big = jnp.float32(3.0e38)
            gidx = jnp.min(jnp.where(rv == vmax[None, :], grow, big), axis=0)
            gather_ref[pl.ds(my, 1)] = jnp.stack([vmax, gidx], axis=0)[None]

            barrier_sem = pltpu.get_barrier_semaphore()
            pl.semaphore_wait(barrier_sem, N_DEV - 1)

            sends = []
            for off in range(1, N_DEV):
                dst = lax.rem(my + off, N_DEV)
                rdma = pltpu.make_async_remote_copy(
                    src_ref=gather_ref.at[my],
                    dst_ref=gather_ref.at[my],
                    send_sem=send_sems.at[off - 1],
                    recv_sem=recv_sems.at[my],
                    device_id=(dst,),
                    device_id_type=pl.DeviceIdType.MESH,
                )
                rdma.start()
                sends.append(rdma)

            for off in range(1, N_DEV):
                src = lax.rem(my - off + N_DEV, N_DEV)
                recv = pltpu.make_async_remote_copy(
                    src_ref=gather_ref.at[my],
                    dst_ref=gather_ref.at[src],
                    send_sem=send_sems.at[0],
                    recv_sem=recv_sems.at[src],
                    device_id=(src,),
                    device_id_type=pl.DeviceIdType.MESH,
                )
                recv.wait_recv()

            for rdma in sends:
                rdma.wait_send()

            vals = gather_ref[:, 0, :]
            idxs = gather_ref[:, 1, :]
            gmax = jnp.max(vals, axis=0)
            gi = jnp.min(jnp.where(vals == gmax[None, :], idxs, big), axis=0)
            out_ref[0, :] = gmax
            out_ref[1, :] = gi

    return pl.pallas_call(
        body,
        grid=(N_BLK,),
        out_shape=jax.ShapeDtypeStruct((2, n), jnp.float32),
        in_specs=[
            pl.BlockSpec((blk, n), lambda i: (i, 0), memory_space=pltpu.VMEM)
        ],
        out_specs=pl.BlockSpec((2, n), lambda i: (0, 0), memory_space=pltpu.VMEM),
        scratch_shapes=[
            pltpu.VMEM((SUB, n), jnp.float32),
            pltpu.VMEM((SUB, n), jnp.int32),
            pltpu.VMEM((N_DEV, 2, n), jnp.float32),
            pltpu.SemaphoreType.DMA((N_DEV - 1,)),
            pltpu.SemaphoreType.DMA((N_DEV,)),
        ],
        compiler_params=pltpu.CompilerParams(collective_id=0),
    )(x)
